# baseline (speedup 1.0000x reference)
import functools

import jax
import jax.numpy as jnp
from jax import lax
from jax.experimental import pallas as pl
from jax.experimental.pallas import tpu as pltpu
from jax.experimental.pallas import tpu_sc as plsc

K_TOP = 200
NMS_THR = 0.5
B = 4
N = 5000
NP = 5120
KP = 208
L = 16
NC = 2
NW = 32
CHUNK = 64
NB = 2


def _nms_body(y1_h, x1_h, y2_h, x2_h, ord_h,
              keep_h, ry1_h, rx1_h, ry2_h, rx2_h, cnt_h,
              vy1, vx1, vy2, vx2, vord,
              ky1, kx1, ky2, kx2, kar,
              okeep, oy1, ox1, oy2, ox2, ocnt):
    c = lax.axis_index("c")
    s = lax.axis_index("s")
    wid = s * NC + c
    b = jnp.minimum(wid, NB - 1)

    pltpu.sync_copy(y1_h.at[b], vy1)
    pltpu.sync_copy(x1_h.at[b], vx1)
    pltpu.sync_copy(y2_h.at[b], vy2)
    pltpu.sync_copy(x2_h.at[b], vx2)
    pltpu.sync_copy(ord_h.at[b], vord)

    zf = jnp.zeros((L,), jnp.float32)
    zi = jnp.zeros((L,), jnp.int32)
    for t in range(KP // L):
        sl = pl.ds(t * L, L)
        ky1[sl] = zf
        kx1[sl] = zf
        ky2[sl] = zf
        kx2[sl] = zf
        kar[sl] = zf
        okeep[sl] = zi
        oy1[sl] = zf
        ox1[sl] = zf
        oy2[sl] = zf
        ox2[sl] = zf

    lanes = lax.iota(jnp.int32, L)
    lane0 = lanes == 0

    def load_cand(p):
        pv = jnp.full((L,), p, jnp.int32)
        idxv = plsc.load_gather(vord, [pv])
        y1c = plsc.load_gather(vy1, [idxv])
        x1c = plsc.load_gather(vx1, [idxv])
        y2c = plsc.load_gather(vy2, [idxv])
        x2c = plsc.load_gather(vx2, [idxv])
        return (idxv, y1c, x1c, y2c, x2c)

    def pos_body(p, state):
        kept, cur = state
        idxv, y1c, x1c, y2c, x2c = cur
        nxt = load_cand(jnp.minimum(p + 1, NP - 1))
        areac = (x2c - x1c) * (y2c - y1c)
        miou0 = jnp.where(areac >= 4.0, jnp.full((L,), -1.0, jnp.float32),
                          jnp.full((L,), 2.0, jnp.float32))

        nk = jnp.where(kept < K_TOP, (kept + (L - 1)) // L, 0)

        def iou_step(t, miou):
            sl = pl.ds(t * L, L)
            a1 = ky1[sl]
            b1 = kx1[sl]
            a2 = ky2[sl]
            b2 = kx2[sl]
            ka = kar[sl]
            q_y1 = jnp.minimum(jnp.maximum(y1c, a1), a2)
            q_x1 = jnp.minimum(jnp.maximum(x1c, b1), b2)
            q_y2 = jnp.minimum(jnp.maximum(y2c, a1), a2)
            q_x2 = jnp.minimum(jnp.maximum(x2c, b1), b2)
            inter = (q_x2 - q_x1) * (q_y2 - q_y1)
            union = areac + ka - inter
            return jnp.maximum(miou, inter / union)

        miou = lax.fori_loop(0, nk, iou_step, miou0)
        take = jnp.logical_and(kept < K_TOP, jnp.max(miou) <= NMS_THR)

        @pl.when(take)
        def _accept():
            kv = jnp.full((L,), kept, jnp.int32)
            plsc.store_scatter(ky1, [kv], y1c, mask=lane0)
            plsc.store_scatter(kx1, [kv], x1c, mask=lane0)
            plsc.store_scatter(ky2, [kv], y2c, mask=lane0)
            plsc.store_scatter(kx2, [kv], x2c, mask=lane0)
            plsc.store_scatter(kar, [kv], areac, mask=lane0)
            plsc.store_scatter(okeep, [kv], idxv, mask=lane0)
            plsc.store_scatter(oy1, [kv], y1c, mask=lane0)
            plsc.store_scatter(ox1, [kv], x1c, mask=lane0)
            plsc.store_scatter(oy2, [kv], y2c, mask=lane0)
            plsc.store_scatter(ox2, [kv], x2c, mask=lane0)

        return (jnp.where(take, kept + 1, kept), nxt)

    def chunk_body(t, state):
        return lax.cond(state[0] < K_TOP,
                        lambda st: lax.fori_loop(t * CHUNK, (t + 1) * CHUNK,
                                                 pos_body, st),
                        lambda st: st,
                        state)

    kept, _ = lax.fori_loop(0, NP // CHUNK, chunk_body,
                            (jnp.int32(0), load_cand(0)))

    ocnt[...] = jnp.full((L,), kept, jnp.int32)

    pltpu.sync_copy(okeep, keep_h.at[wid])
    pltpu.sync_copy(oy1, ry1_h.at[wid])
    pltpu.sync_copy(ox1, rx1_h.at[wid])
    pltpu.sync_copy(oy2, ry2_h.at[wid])
    pltpu.sync_copy(ox2, rx2_h.at[wid])
    pltpu.sync_copy(ocnt, cnt_h.at[wid])


_nms_sc = functools.partial(
    pl.kernel,
    out_type=(
        jax.ShapeDtypeStruct((NW, KP), jnp.int32),
        jax.ShapeDtypeStruct((NW, KP), jnp.float32),
        jax.ShapeDtypeStruct((NW, KP), jnp.float32),
        jax.ShapeDtypeStruct((NW, KP), jnp.float32),
        jax.ShapeDtypeStruct((NW, KP), jnp.float32),
        jax.ShapeDtypeStruct((NW, L), jnp.int32),
    ),
    mesh=plsc.VectorSubcoreMesh(core_axis_name="c", subcore_axis_name="s"),
    scratch_types=[
        pltpu.VMEM((NP,), jnp.float32),
        pltpu.VMEM((NP,), jnp.float32),
        pltpu.VMEM((NP,), jnp.float32),
        pltpu.VMEM((NP,), jnp.float32),
        pltpu.VMEM((NP,), jnp.int32),
        pltpu.VMEM((KP,), jnp.float32),
        pltpu.VMEM((KP,), jnp.float32),
        pltpu.VMEM((KP,), jnp.float32),
        pltpu.VMEM((KP,), jnp.float32),
        pltpu.VMEM((KP,), jnp.float32),
        pltpu.VMEM((KP,), jnp.int32),
        pltpu.VMEM((KP,), jnp.float32),
        pltpu.VMEM((KP,), jnp.float32),
        pltpu.VMEM((KP,), jnp.float32),
        pltpu.VMEM((KP,), jnp.float32),
        pltpu.VMEM((L,), jnp.int32),
    ],
    compiler_params=pltpu.CompilerParams(needs_layout_passes=False),
)(_nms_body)


def kernel(scoress, bboxess):
    pad = ((0, 0), (0, NP - N))
    y1 = jnp.pad(bboxess[:, :, 0], pad)
    x1 = jnp.pad(bboxess[:, :, 1], pad)
    y2 = jnp.pad(bboxess[:, :, 2], pad)
    x2 = jnp.pad(bboxess[:, :, 3], pad)

    outs = []
    for g in range(0, B, NB):
        gs = slice(g, g + NB)
        probs = jax.nn.softmax(scoress[gs], axis=2)
        sc = probs[:, :, 0]
        order_desc = jnp.argsort(sc, axis=1,
                                 stable=True)[:, ::-1].astype(jnp.int32)
        orderp = jnp.pad(order_desc, pad, constant_values=N)
        outs.append(_nms_sc(y1[gs], x1[gs], y2[gs], x2[gs], orderp))

    okeep = jnp.concatenate([o[0][:NB] for o in outs])
    oy1 = jnp.concatenate([o[1][:NB] for o in outs])
    ox1 = jnp.concatenate([o[2][:NB] for o in outs])
    oy2 = jnp.concatenate([o[3][:NB] for o in outs])
    ox2 = jnp.concatenate([o[4][:NB] for o in outs])
    ocnt = jnp.concatenate([o[5][:NB] for o in outs])

    keeps = okeep[:, :K_TOP].astype(jnp.int64)
    counts = ocnt[:, :1].astype(jnp.int64)
    ret = jnp.stack([oy1[:, :K_TOP], ox1[:, :K_TOP],
                     oy2[:, :K_TOP], ox2[:, :K_TOP]], axis=-1)
    return (ret, counts, keeps)


# speedup vs baseline: 1.6234x; 1.6234x over previous
import functools

import jax
import jax.numpy as jnp
from jax import lax
from jax.experimental import pallas as pl
from jax.experimental.pallas import tpu as pltpu
from jax.experimental.pallas import tpu_sc as plsc

K_TOP = 200
NMS_THR = 0.5
B = 4
N = 5000
NP = 5120
KP = 208
L = 16
NC = 2
NW = 32
CHUNK = 64


def _nms_body(y1_h, x1_h, y2_h, x2_h, ord_h,
              keep_h, ry1_h, rx1_h, ry2_h, rx2_h, cnt_h,
              vy1, vx1, vy2, vx2, vord,
              ky1, kx1, ky2, kx2, kar,
              okeep, oy1, ox1, oy2, ox2, ocnt):
    c = lax.axis_index("c")
    s = lax.axis_index("s")
    wid = s * NC + c
    b = jnp.minimum(wid, B - 1)

    pltpu.sync_copy(y1_h.at[b], vy1)
    pltpu.sync_copy(x1_h.at[b], vx1)
    pltpu.sync_copy(y2_h.at[b], vy2)
    pltpu.sync_copy(x2_h.at[b], vx2)
    pltpu.sync_copy(ord_h.at[b], vord)

    zf = jnp.zeros((L,), jnp.float32)
    zi = jnp.zeros((L,), jnp.int32)
    for t in range(KP // L):
        sl = pl.ds(t * L, L)
        ky1[sl] = zf
        kx1[sl] = zf
        ky2[sl] = zf
        kx2[sl] = zf
        kar[sl] = zf
        okeep[sl] = zi
        oy1[sl] = zf
        ox1[sl] = zf
        oy2[sl] = zf
        ox2[sl] = zf

    lanes = lax.iota(jnp.int32, L)
    lane0 = lanes == 0

    def load_cand(p):
        pv = jnp.full((L,), p, jnp.int32)
        idxv = plsc.load_gather(vord, [pv])
        y1c = plsc.load_gather(vy1, [idxv])
        x1c = plsc.load_gather(vx1, [idxv])
        y2c = plsc.load_gather(vy2, [idxv])
        x2c = plsc.load_gather(vx2, [idxv])
        return (idxv, y1c, x1c, y2c, x2c)

    def load_pair(p):
        return load_cand(p) + load_cand(p + 1)

    def accept(kpos, idxv, y1c, x1c, y2c, x2c, areac):
        kv = jnp.full((L,), kpos, jnp.int32)
        plsc.store_scatter(ky1, [kv], y1c, mask=lane0)
        plsc.store_scatter(kx1, [kv], x1c, mask=lane0)
        plsc.store_scatter(ky2, [kv], y2c, mask=lane0)
        plsc.store_scatter(kx2, [kv], x2c, mask=lane0)
        plsc.store_scatter(kar, [kv], areac, mask=lane0)
        plsc.store_scatter(okeep, [kv], idxv, mask=lane0)
        plsc.store_scatter(oy1, [kv], y1c, mask=lane0)
        plsc.store_scatter(ox1, [kv], x1c, mask=lane0)
        plsc.store_scatter(oy2, [kv], y2c, mask=lane0)
        plsc.store_scatter(ox2, [kv], x2c, mask=lane0)

    def pair_body(j, state):
        kept, cur = state
        (iA, y1A, x1A, y2A, x2A, iB, y1B, x1B, y2B, x2B) = cur
        nxt = load_pair(jnp.minimum(2 * j + 2, NP - 2))
        areaA = (x2A - x1A) * (y2A - y1A)
        areaB = (x2B - x1B) * (y2B - y1B)
        neg1 = jnp.full((L,), -1.0, jnp.float32)
        two = jnp.full((L,), 2.0, jnp.float32)
        mA0 = jnp.where(areaA >= 4.0, neg1, two)
        mB0 = jnp.where(areaB >= 4.0, neg1, two)

        nk = jnp.where(kept < K_TOP, (kept + (L - 1)) // L, 0)

        def iou_step(t, ms):
            mA, mB = ms
            sl = pl.ds(t * L, L)
            a1 = ky1[sl]
            b1 = kx1[sl]
            a2 = ky2[sl]
            b2 = kx2[sl]
            ka = kar[sl]
            qa_y1 = jnp.minimum(jnp.maximum(y1A, a1), a2)
            qa_x1 = jnp.minimum(jnp.maximum(x1A, b1), b2)
            qa_y2 = jnp.minimum(jnp.maximum(y2A, a1), a2)
            qa_x2 = jnp.minimum(jnp.maximum(x2A, b1), b2)
            ia = (qa_x2 - qa_x1) * (qa_y2 - qa_y1)
            ua = areaA + ka - ia
            qb_y1 = jnp.minimum(jnp.maximum(y1B, a1), a2)
            qb_x1 = jnp.minimum(jnp.maximum(x1B, b1), b2)
            qb_y2 = jnp.minimum(jnp.maximum(y2B, a1), a2)
            qb_x2 = jnp.minimum(jnp.maximum(x2B, b1), b2)
            ib = (qb_x2 - qb_x1) * (qb_y2 - qb_y1)
            ub = areaB + ka - ib
            return (jnp.maximum(mA, ia / ua), jnp.maximum(mB, ib / ub))

        mA, mB = lax.fori_loop(0, nk, iou_step, (mA0, mB0))
        takeA = jnp.logical_and(kept < K_TOP, jnp.max(mA) <= NMS_THR)
        takeAi = takeA.astype(jnp.int32)

        q_y1 = jnp.minimum(jnp.maximum(y1B, y1A), y2A)
        q_x1 = jnp.minimum(jnp.maximum(x1B, x1A), x2A)
        q_y2 = jnp.minimum(jnp.maximum(y2B, y1A), y2A)
        q_x2 = jnp.minimum(jnp.maximum(x2B, x1A), x2A)
        iab = (q_x2 - q_x1) * (q_y2 - q_y1)
        uab = areaB + areaA - iab
        takeAv = jnp.full((L,), takeAi, jnp.int32) == 1
        mB2 = jnp.where(takeAv, jnp.maximum(mB, iab / uab), mB)
        takeB = jnp.logical_and(kept + takeAi < K_TOP,
                                jnp.max(mB2) <= NMS_THR)

        @pl.when(takeA)
        def _acceptA():
            accept(kept, iA, y1A, x1A, y2A, x2A, areaA)

        @pl.when(takeB)
        def _acceptB():
            accept(kept + takeAi, iB, y1B, x1B, y2B, x2B, areaB)

        return (kept + takeAi + takeB.astype(jnp.int32), nxt)

    def chunk_body(t, state):
        return lax.cond(state[0] < K_TOP,
                        lambda st: lax.fori_loop(t * (CHUNK // 2),
                                                 (t + 1) * (CHUNK // 2),
                                                 pair_body, st),
                        lambda st: st,
                        state)

    kept, _ = lax.fori_loop(0, NP // CHUNK, chunk_body,
                            (jnp.int32(0), load_pair(0)))

    ocnt[...] = jnp.full((L,), kept, jnp.int32)

    pltpu.sync_copy(okeep, keep_h.at[wid])
    pltpu.sync_copy(oy1, ry1_h.at[wid])
    pltpu.sync_copy(ox1, rx1_h.at[wid])
    pltpu.sync_copy(oy2, ry2_h.at[wid])
    pltpu.sync_copy(ox2, rx2_h.at[wid])
    pltpu.sync_copy(ocnt, cnt_h.at[wid])


_nms_sc = functools.partial(
    pl.kernel,
    out_type=(
        jax.ShapeDtypeStruct((NW, KP), jnp.int32),
        jax.ShapeDtypeStruct((NW, KP), jnp.float32),
        jax.ShapeDtypeStruct((NW, KP), jnp.float32),
        jax.ShapeDtypeStruct((NW, KP), jnp.float32),
        jax.ShapeDtypeStruct((NW, KP), jnp.float32),
        jax.ShapeDtypeStruct((NW, L), jnp.int32),
    ),
    mesh=plsc.VectorSubcoreMesh(core_axis_name="c", subcore_axis_name="s"),
    scratch_types=[
        pltpu.VMEM((NP,), jnp.float32),
        pltpu.VMEM((NP,), jnp.float32),
        pltpu.VMEM((NP,), jnp.float32),
        pltpu.VMEM((NP,), jnp.float32),
        pltpu.VMEM((NP,), jnp.int32),
        pltpu.VMEM((KP,), jnp.float32),
        pltpu.VMEM((KP,), jnp.float32),
        pltpu.VMEM((KP,), jnp.float32),
        pltpu.VMEM((KP,), jnp.float32),
        pltpu.VMEM((KP,), jnp.float32),
        pltpu.VMEM((KP,), jnp.int32),
        pltpu.VMEM((KP,), jnp.float32),
        pltpu.VMEM((KP,), jnp.float32),
        pltpu.VMEM((KP,), jnp.float32),
        pltpu.VMEM((KP,), jnp.float32),
        pltpu.VMEM((L,), jnp.int32),
    ],
    compiler_params=pltpu.CompilerParams(needs_layout_passes=False),
)(_nms_body)


def kernel(scoress, bboxess):
    probs = jax.nn.softmax(scoress, axis=2)
    sc = probs[:, :, 0]
    order_desc = jnp.argsort(sc, axis=1, stable=True)[:, ::-1].astype(jnp.int32)

    pad = ((0, 0), (0, NP - N))
    y1 = jnp.pad(bboxess[:, :, 0], pad)
    x1 = jnp.pad(bboxess[:, :, 1], pad)
    y2 = jnp.pad(bboxess[:, :, 2], pad)
    x2 = jnp.pad(bboxess[:, :, 3], pad)
    orderp = jnp.pad(order_desc, pad, constant_values=N)

    okeep, oy1, ox1, oy2, ox2, ocnt = _nms_sc(y1, x1, y2, x2, orderp)

    keeps = okeep[:B, :K_TOP].astype(jnp.int64)
    counts = ocnt[:B, :1].astype(jnp.int64)
    ret = jnp.stack([oy1[:B, :K_TOP], ox1[:B, :K_TOP],
                     oy2[:B, :K_TOP], ox2[:B, :K_TOP]], axis=-1)
    return (ret, counts, keeps)


# speedup vs baseline: 1.6249x; 1.0010x over previous
import functools

import jax
import jax.numpy as jnp
from jax import lax
from jax.experimental import pallas as pl
from jax.experimental.pallas import tpu as pltpu
from jax.experimental.pallas import tpu_sc as plsc

K_TOP = 200
NMS_THR = 0.5
B = 4
N = 5000
NP = 5120
KP = 208
L = 16
NC = 2
NW = 32
CHUNK = 64


def _nms_body(y1_h, x1_h, y2_h, x2_h, ord_h,
              keep_h, ry1_h, rx1_h, ry2_h, rx2_h, cnt_h,
              vy1, vx1, vy2, vx2, vord,
              ky1, kx1, ky2, kx2, kar,
              okeep, oy1, ox1, oy2, ox2, ocnt):
    c = lax.axis_index("c")
    s = lax.axis_index("s")
    wid = s * NC + c
    b = jnp.minimum(wid, B - 1)

    pltpu.sync_copy(y1_h.at[b], vy1)
    pltpu.sync_copy(x1_h.at[b], vx1)
    pltpu.sync_copy(y2_h.at[b], vy2)
    pltpu.sync_copy(x2_h.at[b], vx2)
    pltpu.sync_copy(ord_h.at[b], vord)

    zf = jnp.zeros((L,), jnp.float32)
    zi = jnp.zeros((L,), jnp.int32)
    for t in range(KP // L):
        sl = pl.ds(t * L, L)
        ky1[sl] = zf
        kx1[sl] = zf
        ky2[sl] = zf
        kx2[sl] = zf
        kar[sl] = zf
        okeep[sl] = zi
        oy1[sl] = zf
        ox1[sl] = zf
        oy2[sl] = zf
        ox2[sl] = zf

    lanes = lax.iota(jnp.int32, L)
    lane0 = lanes == 0

    def load_cand(p):
        pv = jnp.full((L,), p, jnp.int32)
        idxv = plsc.load_gather(vord, [pv])
        y1c = plsc.load_gather(vy1, [idxv])
        x1c = plsc.load_gather(vx1, [idxv])
        y2c = plsc.load_gather(vy2, [idxv])
        x2c = plsc.load_gather(vx2, [idxv])
        return (idxv, y1c, x1c, y2c, x2c)

    def load_pair(p):
        return load_cand(p) + load_cand(p + 1)

    def accept(kpos, idxv, y1c, x1c, y2c, x2c, areac):
        kv = jnp.full((L,), kpos, jnp.int32)
        plsc.store_scatter(ky1, [kv], y1c, mask=lane0)
        plsc.store_scatter(kx1, [kv], x1c, mask=lane0)
        plsc.store_scatter(ky2, [kv], y2c, mask=lane0)
        plsc.store_scatter(kx2, [kv], x2c, mask=lane0)
        plsc.store_scatter(kar, [kv], areac, mask=lane0)
        plsc.store_scatter(okeep, [kv], idxv, mask=lane0)
        plsc.store_scatter(oy1, [kv], y1c, mask=lane0)
        plsc.store_scatter(ox1, [kv], x1c, mask=lane0)
        plsc.store_scatter(oy2, [kv], y2c, mask=lane0)
        plsc.store_scatter(ox2, [kv], x2c, mask=lane0)

    def pair_body(j, state):
        kept, cur = state
        (iA, y1A, x1A, y2A, x2A, iB, y1B, x1B, y2B, x2B) = cur
        nxt = load_pair(jnp.minimum(2 * j + 2, NP - 2))
        areaA = (x2A - x1A) * (y2A - y1A)
        areaB = (x2B - x1B) * (y2B - y1B)
        neg1 = jnp.full((L,), -1.0, jnp.float32)
        two = jnp.full((L,), 2.0, jnp.float32)
        mA0 = jnp.where(areaA >= 4.0, neg1, two)
        mB0 = jnp.where(areaB >= 4.0, neg1, two)

        nk = jnp.where(kept < K_TOP, (kept + (L - 1)) // L, 0)

        def iou_step(t, ms):
            mA, mB = ms
            sl = pl.ds(t * L, L)
            a1 = ky1[sl]
            b1 = kx1[sl]
            a2 = ky2[sl]
            b2 = kx2[sl]
            ka = kar[sl]
            qa_y1 = jnp.minimum(jnp.maximum(y1A, a1), a2)
            qa_x1 = jnp.minimum(jnp.maximum(x1A, b1), b2)
            qa_y2 = jnp.minimum(jnp.maximum(y2A, a1), a2)
            qa_x2 = jnp.minimum(jnp.maximum(x2A, b1), b2)
            ia = (qa_x2 - qa_x1) * (qa_y2 - qa_y1)
            ua = areaA + ka - ia
            qb_y1 = jnp.minimum(jnp.maximum(y1B, a1), a2)
            qb_x1 = jnp.minimum(jnp.maximum(x1B, b1), b2)
            qb_y2 = jnp.minimum(jnp.maximum(y2B, a1), a2)
            qb_x2 = jnp.minimum(jnp.maximum(x2B, b1), b2)
            ib = (qb_x2 - qb_x1) * (qb_y2 - qb_y1)
            ub = areaB + ka - ib
            return (jnp.maximum(mA, ia / ua), jnp.maximum(mB, ib / ub))

        mA, mB = lax.fori_loop(0, nk, iou_step, (mA0, mB0))
        takeA = jnp.logical_and(kept < K_TOP, jnp.max(mA) <= NMS_THR)
        takeAi = takeA.astype(jnp.int32)

        q_y1 = jnp.minimum(jnp.maximum(y1B, y1A), y2A)
        q_x1 = jnp.minimum(jnp.maximum(x1B, x1A), x2A)
        q_y2 = jnp.minimum(jnp.maximum(y2B, y1A), y2A)
        q_x2 = jnp.minimum(jnp.maximum(x2B, x1A), x2A)
        iab = (q_x2 - q_x1) * (q_y2 - q_y1)
        uab = areaB + areaA - iab
        okAB = jnp.logical_or(jnp.logical_not(takeA),
                              jnp.max(iab / uab) <= NMS_THR)
        takeB = jnp.logical_and(
            kept + takeAi < K_TOP,
            jnp.logical_and(jnp.max(mB) <= NMS_THR, okAB))

        @pl.when(takeA)
        def _acceptA():
            accept(kept, iA, y1A, x1A, y2A, x2A, areaA)

        @pl.when(takeB)
        def _acceptB():
            accept(kept + takeAi, iB, y1B, x1B, y2B, x2B, areaB)

        return (kept + takeAi + takeB.astype(jnp.int32), nxt)

    def chunk_body(t, state):
        def run(st):
            st2 = (st[0], load_pair(t * CHUNK))
            return lax.fori_loop(t * (CHUNK // 2), (t + 1) * (CHUNK // 2),
                                 pair_body, st2)
        return lax.cond(state[0] < K_TOP, run, lambda st: st, state)

    kept, _ = lax.fori_loop(0, NP // CHUNK, chunk_body,
                            (jnp.int32(0), load_pair(0)))

    ocnt[...] = jnp.full((L,), kept, jnp.int32)

    pltpu.sync_copy(okeep, keep_h.at[wid])
    pltpu.sync_copy(oy1, ry1_h.at[wid])
    pltpu.sync_copy(ox1, rx1_h.at[wid])
    pltpu.sync_copy(oy2, ry2_h.at[wid])
    pltpu.sync_copy(ox2, rx2_h.at[wid])
    pltpu.sync_copy(ocnt, cnt_h.at[wid])


_nms_sc = functools.partial(
    pl.kernel,
    out_type=(
        jax.ShapeDtypeStruct((NW, KP), jnp.int32),
        jax.ShapeDtypeStruct((NW, KP), jnp.float32),
        jax.ShapeDtypeStruct((NW, KP), jnp.float32),
        jax.ShapeDtypeStruct((NW, KP), jnp.float32),
        jax.ShapeDtypeStruct((NW, KP), jnp.float32),
        jax.ShapeDtypeStruct((NW, L), jnp.int32),
    ),
    mesh=plsc.VectorSubcoreMesh(core_axis_name="c", subcore_axis_name="s"),
    scratch_types=[
        pltpu.VMEM((NP,), jnp.float32),
        pltpu.VMEM((NP,), jnp.float32),
        pltpu.VMEM((NP,), jnp.float32),
        pltpu.VMEM((NP,), jnp.float32),
        pltpu.VMEM((NP,), jnp.int32),
        pltpu.VMEM((KP,), jnp.float32),
        pltpu.VMEM((KP,), jnp.float32),
        pltpu.VMEM((KP,), jnp.float32),
        pltpu.VMEM((KP,), jnp.float32),
        pltpu.VMEM((KP,), jnp.float32),
        pltpu.VMEM((KP,), jnp.int32),
        pltpu.VMEM((KP,), jnp.float32),
        pltpu.VMEM((KP,), jnp.float32),
        pltpu.VMEM((KP,), jnp.float32),
        pltpu.VMEM((KP,), jnp.float32),
        pltpu.VMEM((L,), jnp.int32),
    ],
    compiler_params=pltpu.CompilerParams(needs_layout_passes=False),
)(_nms_body)


def kernel(scoress, bboxess):
    probs = jax.nn.softmax(scoress, axis=2)
    sc = probs[:, :, 0]
    order_desc = jnp.argsort(sc, axis=1, stable=True)[:, ::-1].astype(jnp.int32)

    pad = ((0, 0), (0, NP - N))
    y1 = jnp.pad(bboxess[:, :, 0], pad)
    x1 = jnp.pad(bboxess[:, :, 1], pad)
    y2 = jnp.pad(bboxess[:, :, 2], pad)
    x2 = jnp.pad(bboxess[:, :, 3], pad)
    orderp = jnp.pad(order_desc, pad, constant_values=N)

    okeep, oy1, ox1, oy2, ox2, ocnt = _nms_sc(y1, x1, y2, x2, orderp)

    keeps = okeep[:B, :K_TOP].astype(jnp.int64)
    counts = ocnt[:B, :1].astype(jnp.int64)
    ret = jnp.stack([oy1[:B, :K_TOP], ox1[:B, :K_TOP],
                     oy2[:B, :K_TOP], ox2[:B, :K_TOP]], axis=-1)
    return (ret, counts, keeps)
